# Initial kernel scaffold; baseline (speedup 1.0000x reference)
import jax
import jax.numpy as jnp
from jax.experimental import pallas as pl


def kernel(x, embedding):
    raise NotImplementedError("write your pallas kernel here")


# speedup vs baseline: 1.7221x; 1.7221x over previous
import jax
import jax.numpy as jnp
from jax.experimental import pallas as pl


_BS = 512


def _add_kernel(x_ref, emb_ref, o_ref):
    o_ref[...] = x_ref[...] + emb_ref[...][None, :, :]


def kernel(x, embedding):
    batch, seq_len, d_model = x.shape
    bs = _BS if seq_len % _BS == 0 else seq_len
    grid = (seq_len // bs,)
    return pl.pallas_call(
        _add_kernel,
        grid=grid,
        in_specs=[
            pl.BlockSpec((batch, bs, d_model), lambda s: (0, s, 0)),
            pl.BlockSpec((bs, d_model), lambda s: (s, 0)),
        ],
        out_specs=pl.BlockSpec((batch, bs, d_model), lambda s: (0, s, 0)),
        out_shape=jax.ShapeDtypeStruct((batch, seq_len, d_model), x.dtype),
    )(x, embedding)


# speedup vs baseline: 1.7232x; 1.0007x over previous
import jax
import jax.numpy as jnp
from jax.experimental import pallas as pl


_BS = 256


def _add_kernel(x_ref, emb_ref, o_ref):
    o_ref[...] = x_ref[...] + emb_ref[...][None, :, :]


def kernel(x, embedding):
    batch, seq_len, d_model = x.shape
    bs = _BS if seq_len % _BS == 0 else seq_len
    grid = (seq_len // bs,)
    return pl.pallas_call(
        _add_kernel,
        grid=grid,
        in_specs=[
            pl.BlockSpec((batch, bs, d_model), lambda s: (0, s, 0)),
            pl.BlockSpec((bs, d_model), lambda s: (s, 0)),
        ],
        out_specs=pl.BlockSpec((batch, bs, d_model), lambda s: (0, s, 0)),
        out_shape=jax.ShapeDtypeStruct((batch, seq_len, d_model), x.dtype),
    )(x, embedding)
